# baseline (speedup 1.0000x reference)
import functools

import jax
import jax.numpy as jnp
from jax import lax
from jax.experimental import pallas as pl
from jax.experimental.pallas import tpu as pltpu
from jax.experimental.pallas import tpu_sc as plsc

R = 128
C = 32768
L = 16
NC = 2
NS = 16
NW = NC * NS
ROWS_PER_W = R // NW
CHUNKS = C // L


def _permute(v, perm):
    dnums = lax.GatherDimensionNumbers(
        offset_dims=(), collapsed_slice_dims=(0,), start_index_map=(0,))
    return lax.gather(v, perm[:, None], dnums, (1,),
                      mode=lax.GatherScatterMode.PROMISE_IN_BOUNDS)


def _body(x_hbm, out_hbm, in_v, zero_v, hot_v):
    cid = lax.axis_index("c")
    sid = lax.axis_index("s")
    wid = sid * NC + cid

    lanes = lax.iota(jnp.int32, 16)
    zero = jnp.zeros((L,), jnp.float32)
    one = jnp.ones((L,), jnp.float32)

    def zbody(i, carry):
        zero_v[pl.ds(i * L, L)] = zero
        return carry

    lax.fori_loop(0, CHUNKS, zbody, 0, unroll=8)

    for r in range(ROWS_PER_W):
        row = wid * ROWS_PER_W + r
        pltpu.sync_copy(x_hbm.at[row], in_v)

        def scan(i, carry):
            mvec, ivec = carry
            v = in_v[pl.ds(i * L, L)]
            idx = lanes + i * L
            upd = v > mvec
            return jnp.where(upd, v, mvec), jnp.where(upd, idx, ivec)

        mvec, ivec = lax.fori_loop(
            0, CHUNKS, scan,
            (jnp.full((L,), -jnp.inf, jnp.float32),
             jnp.zeros((L,), jnp.int32)),
            unroll=8,
        )

        for s in (8, 4, 2, 1):
            perm = lanes ^ s
            om = _permute(mvec, perm)
            oi = _permute(ivec, perm)
            take = (om > mvec) | ((om == mvec) & (oi < ivec))
            mvec = jnp.where(take, om, mvec)
            ivec = jnp.where(take, oi, ivec)

        best = ivec[0]
        blk = best // L
        lane = best - blk * L

        hot_v[...] = jnp.where(lanes == lane, one, zero)
        pltpu.sync_copy(zero_v, out_hbm.at[row])
        pltpu.sync_copy(hot_v, out_hbm.at[row, pl.ds(blk * L, L)])


@jax.jit
def kernel(x):
    mesh = plsc.VectorSubcoreMesh(core_axis_name="c", subcore_axis_name="s")
    f = pl.kernel(
        _body,
        mesh=mesh,
        out_type=jax.ShapeDtypeStruct((R, C), jnp.float32),
        scratch_types=[
            pltpu.VMEM((C,), jnp.float32),
            pltpu.VMEM((C,), jnp.float32),
            pltpu.VMEM((L,), jnp.float32),
        ],
    )
    return f(x)


# speedup vs baseline: 1.2572x; 1.2572x over previous
import functools

import jax
import jax.numpy as jnp
from jax import lax
from jax.experimental import pallas as pl
from jax.experimental.pallas import tpu as pltpu
from jax.experimental.pallas import tpu_sc as plsc

R = 128
C = 32768
L = 16
NC = 2
NS = 16
NW = NC * NS
ROWS_PER_W = R // NW
ACC = 8
CHUNKS = C // L
STEPS = CHUNKS // ACC


def _permute(v, perm):
    dnums = lax.GatherDimensionNumbers(
        offset_dims=(), collapsed_slice_dims=(0,), start_index_map=(0,))
    return lax.gather(v, perm[:, None], dnums, (1,),
                      mode=lax.GatherScatterMode.PROMISE_IN_BOUNDS)


def _body(x_hbm, out_hbm, in0, in1, zero_v, hot4, sem_in, sem_z, sem_p):
    cid = lax.axis_index("c")
    sid = lax.axis_index("s")
    wid = sid * NC + cid
    row0 = wid * ROWS_PER_W

    lanes = lax.iota(jnp.int32, 16)
    zero = jnp.zeros((L,), jnp.float32)
    one = jnp.ones((L,), jnp.float32)
    neg_inf = jnp.full((L,), -jnp.inf, jnp.float32)
    izero = jnp.zeros((L,), jnp.int32)

    def zbody(i, carry):
        zero_v[pl.ds(i * L, L)] = zero
        return carry

    lax.fori_loop(0, CHUNKS, zbody, 0, unroll=8)

    zcopies = [pltpu.async_copy(zero_v, out_hbm.at[row0 + r], sem_z)
               for r in range(ROWS_PER_W)]

    in_bufs = [in0, in1]
    first = pltpu.async_copy(x_hbm.at[row0], in0, sem_in)
    pending = {0: first}

    pcopies = []
    for r in range(ROWS_PER_W):
        buf = in_bufs[r % 2]
        pending.pop(r).wait()
        if r + 1 < ROWS_PER_W:
            pending[r + 1] = pltpu.async_copy(
                x_hbm.at[row0 + r + 1], in_bufs[(r + 1) % 2], sem_in)

        def scan(i, carry):
            ms, ivs = list(carry[:ACC]), list(carry[ACC:])
            base = i * ACC
            for k in range(ACC):
                c = base + k
                v = buf[pl.ds(c * L, L)]
                idx = lanes + c * L
                upd = v > ms[k]
                ms[k] = jnp.where(upd, v, ms[k])
                ivs[k] = jnp.where(upd, idx, ivs[k])
            return tuple(ms) + tuple(ivs)

        init = tuple([neg_inf] * ACC) + tuple([izero] * ACC)
        res = lax.fori_loop(0, STEPS, scan, init, unroll=2)
        ms, ivs = list(res[:ACC]), list(res[ACC:])

        stride = ACC
        while stride > 1:
            stride //= 2
            for k in range(stride):
                om, oi = ms[k + stride], ivs[k + stride]
                take = (om > ms[k]) | ((om == ms[k]) & (oi < ivs[k]))
                ms[k] = jnp.where(take, om, ms[k])
                ivs[k] = jnp.where(take, oi, ivs[k])
        mvec, ivec = ms[0], ivs[0]
        for s in (8, 4, 2, 1):
            perm = lanes ^ s
            om = _permute(mvec, perm)
            oi = _permute(ivec, perm)
            take = (om > mvec) | ((om == mvec) & (oi < ivec))
            mvec = jnp.where(take, om, mvec)
            ivec = jnp.where(take, oi, ivec)

        best = ivec[0]
        blk = best // L
        lane = best - blk * L
        hot4[pl.ds(r * L, L)] = jnp.where(lanes == lane, one, zero)

        if r == 0:
            for zc in zcopies:
                zc.wait()
        pcopies.append(pltpu.async_copy(
            hot4.at[pl.ds(r * L, L)],
            out_hbm.at[row0 + r, pl.ds(blk * L, L)], sem_p))

    for pc in pcopies:
        pc.wait()


@jax.jit
def kernel(x):
    mesh = plsc.VectorSubcoreMesh(core_axis_name="c", subcore_axis_name="s")
    f = pl.kernel(
        _body,
        mesh=mesh,
        out_type=jax.ShapeDtypeStruct((R, C), jnp.float32),
        scratch_types=[
            pltpu.VMEM((C,), jnp.float32),
            pltpu.VMEM((C,), jnp.float32),
            pltpu.VMEM((C,), jnp.float32),
            pltpu.VMEM((ROWS_PER_W * L,), jnp.float32),
            pltpu.SemaphoreType.DMA,
            pltpu.SemaphoreType.DMA,
            pltpu.SemaphoreType.DMA,
        ],
    )
    return f(x)
